# baseline (speedup 1.0000x reference)
import jax
import jax.numpy as jnp
from jax import lax
from jax.experimental import pallas as pl
from jax.experimental.pallas import tpu as pltpu
from jax.experimental.pallas import tpu_sc as plsc

N = 10000
E = 160000
D = 256
HALF = 128
G = 64

ROW_BLK = 2000
N_BLKS = N // ROW_BLK

NUM_TILES = 16
EDGE_CHUNK = 80
EDGES_PER_TILE = E // NUM_TILES
CHUNKS_PER_TILE = EDGES_PER_TILE // EDGE_CHUNK
ROWS_PER_TILE = 624
TAIL_ROWS = N - NUM_TILES * ROWS_PER_TILE


def _dense1_body(x_ref, wrel_ref, wroot_ref, b_ref, yl_ref, yr_ref, r_ref):
    x = x_ref[...].astype(jnp.bfloat16)
    y = lax.dot_general(x, wrel_ref[...].astype(jnp.bfloat16),
                        (((1,), (1,)), ((), ())),
                        preferred_element_type=jnp.float32)
    yl_ref[...] = y[:, :HALF]
    yr_ref[...] = y[:, HALF:]
    r_ref[...] = lax.dot_general(
        x, wroot_ref[...].astype(jnp.bfloat16), (((1,), (1,)), ((), ())),
        preferred_element_type=jnp.float32) + b_ref[...]


def _dense1(x, W_rel, W_root, b):
    return pl.pallas_call(
        _dense1_body,
        grid=(N_BLKS,),
        in_specs=[
            pl.BlockSpec((ROW_BLK, D), lambda i: (i, 0)),
            pl.BlockSpec((D, D), lambda i: (0, 0)),
            pl.BlockSpec((D, D), lambda i: (0, 0)),
            pl.BlockSpec((1, D), lambda i: (0, 0)),
        ],
        out_specs=[
            pl.BlockSpec((ROW_BLK, HALF), lambda i: (i, 0)),
            pl.BlockSpec((ROW_BLK, HALF), lambda i: (i, 0)),
            pl.BlockSpec((ROW_BLK, D), lambda i: (i, 0)),
        ],
        out_shape=[
            jax.ShapeDtypeStruct((N, HALF), jnp.float32),
            jax.ShapeDtypeStruct((N, HALF), jnp.float32),
            jax.ShapeDtypeStruct((N, D), jnp.float32),
        ],
    )(x, W_rel, W_root, b)


def _dense2_body(r_ref, al_ref, ar_ref, wrel_ref, wroot_ref, b_ref,
                 yl_ref, yr_ref, r2_ref):
    h = jnp.concatenate([al_ref[...], ar_ref[...]], axis=1) + r_ref[...]
    h = jnp.maximum(h, 0.0).astype(jnp.bfloat16)
    y = lax.dot_general(h, wrel_ref[...].astype(jnp.bfloat16),
                        (((1,), (1,)), ((), ())),
                        preferred_element_type=jnp.float32)
    yl_ref[...] = y[:, :HALF]
    yr_ref[...] = y[:, HALF:]
    r2_ref[...] = lax.dot_general(
        h, wroot_ref[...].astype(jnp.bfloat16), (((1,), (1,)), ((), ())),
        preferred_element_type=jnp.float32) + b_ref[...]


def _dense2(r, al, ar, W_rel, W_root, b):
    return pl.pallas_call(
        _dense2_body,
        grid=(N_BLKS,),
        in_specs=[
            pl.BlockSpec((ROW_BLK, D), lambda i: (i, 0)),
            pl.BlockSpec((ROW_BLK, HALF), lambda i: (i, 0)),
            pl.BlockSpec((ROW_BLK, HALF), lambda i: (i, 0)),
            pl.BlockSpec((D, D), lambda i: (0, 0)),
            pl.BlockSpec((D, D), lambda i: (0, 0)),
            pl.BlockSpec((1, D), lambda i: (0, 0)),
        ],
        out_specs=[
            pl.BlockSpec((ROW_BLK, HALF), lambda i: (i, 0)),
            pl.BlockSpec((ROW_BLK, HALF), lambda i: (i, 0)),
            pl.BlockSpec((ROW_BLK, D), lambda i: (i, 0)),
        ],
        out_shape=[
            jax.ShapeDtypeStruct((N, HALF), jnp.float32),
            jax.ShapeDtypeStruct((N, HALF), jnp.float32),
            jax.ShapeDtypeStruct((N, D), jnp.float32),
        ],
    )(r, al, ar, W_rel, W_root, b)


def _pool_body(r_ref, al_ref, ar_ref, batch_ref, o_ref):
    i = pl.program_id(0)
    h = jnp.concatenate([al_ref[...], ar_ref[...]], axis=1) + r_ref[...]
    h = jnp.maximum(h, 0.0)
    b = batch_ref[0, 0, :]
    mask = (b[:, None] == lax.broadcasted_iota(jnp.int32, (ROW_BLK, G), 1)
            ).astype(jnp.bfloat16)
    contrib = lax.dot_general(mask, h.astype(jnp.bfloat16),
                              (((0,), (0,)), ((), ())),
                              preferred_element_type=jnp.float32)

    @pl.when(i == 0)
    def _():
        o_ref[...] = contrib

    @pl.when(i != 0)
    def _():
        o_ref[...] += contrib


def _pool(r, al, ar, batch3):
    return pl.pallas_call(
        _pool_body,
        grid=(N_BLKS,),
        in_specs=[
            pl.BlockSpec((ROW_BLK, D), lambda i: (i, 0)),
            pl.BlockSpec((ROW_BLK, HALF), lambda i: (i, 0)),
            pl.BlockSpec((ROW_BLK, HALF), lambda i: (i, 0)),
            pl.BlockSpec((1, 1, ROW_BLK), lambda i: (i, 0, 0)),
        ],
        out_specs=pl.BlockSpec((G, D), lambda i: (0, 0)),
        out_shape=jax.ShapeDtypeStruct((G, D), jnp.float32),
    )(r, al, ar, batch3)


SEG_A = 64
SEG_B = CHUNKS_PER_TILE - SEG_A


def _sc_scatter_body(yl_hbm, yr_hbm, src_hbm, dst_hbm, al_hbm, ar_hbm,
                     acc, b0, b1, b2, src2v, dst2v,
                     gs0, gs1, gs2, ss0, ss1, ss2):
    c = lax.axis_index("c")
    t = lax.axis_index("s")
    bufs = (b0, b1, b2)
    gs = (gs0, gs1, gs2)
    ss = (ss0, ss1, ss2)

    pltpu.sync_copy(src_hbm.at[t, pl.ds(0, SEG_A)], src2v)
    pltpu.sync_copy(dst_hbm.at[t, pl.ds(0, SEG_A)], dst2v)

    zero16 = jnp.zeros((16,), jnp.float32)

    def zrow(i, carry):
        for k in range(HALF // 16):
            b0[i, pl.ds(k * 16, 16)] = zero16
            b1[i, pl.ds(k * 16, 16)] = zero16
        return carry

    lax.fori_loop(0, EDGE_CHUNK, zrow, 0)

    row0 = t * ROWS_PER_TILE
    for bi in range(ROWS_PER_TILE // EDGE_CHUNK):
        pltpu.sync_copy(b0 if bi % 2 == 0 else b1,
                        acc.at[pl.ds(row0 + bi * EDGE_CHUNK, EDGE_CHUNK)])
    pltpu.sync_copy(b1.at[pl.ds(0, ROWS_PER_TILE % EDGE_CHUNK)],
                    acc.at[pl.ds(row0 + 560, ROWS_PER_TILE % EDGE_CHUNK)])

    @pl.when(t == NUM_TILES - 1)
    def _():
        pltpu.sync_copy(b0.at[pl.ds(0, TAIL_ROWS)],
                        acc.at[pl.ds(NUM_TILES * ROWS_PER_TILE, TAIL_ROWS)])

    plsc.subcore_barrier()

    def gath(l, b):
        idx = src2v.at[l]

        @pl.when(c == 0)
        def _():
            pltpu.async_copy(yl_hbm.at[idx], bufs[b], gs[b])

        @pl.when(c == 1)
        def _():
            pltpu.async_copy(yr_hbm.at[idx], bufs[b], gs[b])

    def gw(b):
        pltpu.make_async_copy(
            yl_hbm.at[src2v.at[0]], bufs[b], gs[b]).wait()

    def sf(l, b):
        pltpu.async_copy(bufs[b], acc.at[dst2v.at[l]], ss[b], add=True)

    def sw(b):
        pltpu.make_async_copy(bufs[b], acc.at[dst2v.at[0]], ss[b]).wait()

    def step(l, b, refire):
        gw(b)
        sf(l, b)
        if refire:
            bp = (b + 2) % 3
            sw(bp)
            gath(l + 2, bp)

    def run_segment(n, n_groups, tail_refire_steps, tail_steps):
        gath(0, 0)
        gath(1, 1)
        gath(2, 2)
        step(0, 0, False)

        def grp(g, carry):
            l = 3 * g
            step(l + 1, 1, True)
            step(l + 2, 2, True)
            step(l + 3, 0, True)
            return carry

        lax.fori_loop(0, n_groups, grp, 0)
        for l in tail_refire_steps:
            step(l, l % 3, True)
        for l in tail_steps:
            step(l, l % 3, False)
        sw(0)
        sw(1)
        sw(2)

    run_segment(SEG_A, 20, [61], [62, 63])

    pltpu.sync_copy(src_hbm.at[t, pl.ds(SEG_A, SEG_B)],
                    src2v.at[pl.ds(0, SEG_B)])
    pltpu.sync_copy(dst_hbm.at[t, pl.ds(SEG_A, SEG_B)],
                    dst2v.at[pl.ds(0, SEG_B)])

    run_segment(SEG_B, 19, [58], [59, 60])

    plsc.subcore_barrier()

    tail0 = NUM_TILES * ROWS_PER_TILE

    @pl.when(c == 0)
    def _():
        pltpu.sync_copy(acc.at[pl.ds(row0, ROWS_PER_TILE)],
                        al_hbm.at[pl.ds(row0, ROWS_PER_TILE)])

        @pl.when(t == NUM_TILES - 1)
        def _():
            pltpu.sync_copy(acc.at[pl.ds(tail0, TAIL_ROWS)],
                            al_hbm.at[pl.ds(tail0, TAIL_ROWS)])

    @pl.when(c == 1)
    def _():
        pltpu.sync_copy(acc.at[pl.ds(row0, ROWS_PER_TILE)],
                        ar_hbm.at[pl.ds(row0, ROWS_PER_TILE)])

        @pl.when(t == NUM_TILES - 1)
        def _():
            pltpu.sync_copy(acc.at[pl.ds(tail0, TAIL_ROWS)],
                            ar_hbm.at[pl.ds(tail0, TAIL_ROWS)])


_SC_MESH = plsc.VectorSubcoreMesh(core_axis_name="c", subcore_axis_name="s",
                                  num_cores=2, num_subcores=NUM_TILES)

_sc_scatter = pl.kernel(
    _sc_scatter_body,
    out_type=[
        jax.ShapeDtypeStruct((N, HALF), jnp.float32),
        jax.ShapeDtypeStruct((N, HALF), jnp.float32),
    ],
    mesh=_SC_MESH,
    scratch_types=[
        pltpu.VMEM_SHARED((N, HALF), jnp.float32),
        pltpu.VMEM((EDGE_CHUNK, HALF), jnp.float32),
        pltpu.VMEM((EDGE_CHUNK, HALF), jnp.float32),
        pltpu.VMEM((EDGE_CHUNK, HALF), jnp.float32),
        pltpu.VMEM((SEG_A, EDGE_CHUNK), jnp.int32),
        pltpu.VMEM((SEG_A, EDGE_CHUNK), jnp.int32),
        pltpu.SemaphoreType.DMA,
        pltpu.SemaphoreType.DMA,
        pltpu.SemaphoreType.DMA,
        pltpu.SemaphoreType.DMA,
        pltpu.SemaphoreType.DMA,
        pltpu.SemaphoreType.DMA,
    ],
)


def kernel(x, edge_index, batch, W1_rel, b1, W1_root, W2_rel, b2, W2_root):
    src = edge_index[0].astype(jnp.int32).reshape(
        NUM_TILES, CHUNKS_PER_TILE, EDGE_CHUNK)
    dst = edge_index[1].astype(jnp.int32).reshape(
        NUM_TILES, CHUNKS_PER_TILE, EDGE_CHUNK)
    b1r = b1.reshape(1, D)
    b2r = b2.reshape(1, D)
    batch3 = batch.astype(jnp.int32).reshape(N_BLKS, 1, ROW_BLK)

    yl1, yr1, r1 = _dense1(x, W1_rel, W1_root, b1r)
    al1, ar1 = _sc_scatter(yl1, yr1, src, dst)
    yl2, yr2, r2 = _dense2(r1, al1, ar1, W2_rel, W2_root, b2r)
    ql, qr = _sc_scatter(yl2, yr2, src, dst)
    return _pool(r2, ql, qr, batch3)


# speedup vs baseline: 1.0504x; 1.0504x over previous
import jax
import jax.numpy as jnp
from jax import lax
from jax.experimental import pallas as pl
from jax.experimental.pallas import tpu as pltpu
from jax.experimental.pallas import tpu_sc as plsc

N = 10000
E = 160000
D = 256
HALF = 128
G = 64

ROW_BLK = 1000
N_BLKS = N // ROW_BLK

NUM_TILES = 16
EDGE_CHUNK = 80
EDGES_PER_TILE = E // NUM_TILES
CHUNKS_PER_TILE = EDGES_PER_TILE // EDGE_CHUNK
ROWS_PER_TILE = 624
TAIL_ROWS = N - NUM_TILES * ROWS_PER_TILE


def _dense1_body(x_ref, wrel_ref, wroot_ref, b_ref, yl_ref, yr_ref, r_ref):
    x = x_ref[...].astype(jnp.bfloat16)
    y = lax.dot_general(x, wrel_ref[...].astype(jnp.bfloat16),
                        (((1,), (1,)), ((), ())),
                        preferred_element_type=jnp.float32)
    yl_ref[...] = y[:, :HALF]
    yr_ref[...] = y[:, HALF:]
    r_ref[...] = lax.dot_general(
        x, wroot_ref[...].astype(jnp.bfloat16), (((1,), (1,)), ((), ())),
        preferred_element_type=jnp.float32) + b_ref[...]


def _dense1(x, W_rel, W_root, b):
    return pl.pallas_call(
        _dense1_body,
        grid=(N_BLKS,),
        in_specs=[
            pl.BlockSpec((ROW_BLK, D), lambda i: (i, 0)),
            pl.BlockSpec((D, D), lambda i: (0, 0)),
            pl.BlockSpec((D, D), lambda i: (0, 0)),
            pl.BlockSpec((1, D), lambda i: (0, 0)),
        ],
        out_specs=[
            pl.BlockSpec((ROW_BLK, HALF), lambda i: (i, 0)),
            pl.BlockSpec((ROW_BLK, HALF), lambda i: (i, 0)),
            pl.BlockSpec((ROW_BLK, D), lambda i: (i, 0)),
        ],
        out_shape=[
            jax.ShapeDtypeStruct((N, HALF), jnp.float32),
            jax.ShapeDtypeStruct((N, HALF), jnp.float32),
            jax.ShapeDtypeStruct((N, D), jnp.float32),
        ],
    )(x, W_rel, W_root, b)


def _dense2_body(r_ref, al_ref, ar_ref, wrel_ref, wroot_ref, b_ref,
                 yl_ref, yr_ref, r2_ref):
    h = jnp.concatenate([al_ref[...], ar_ref[...]], axis=1) + r_ref[...]
    h = jnp.maximum(h, 0.0).astype(jnp.bfloat16)
    y = lax.dot_general(h, wrel_ref[...].astype(jnp.bfloat16),
                        (((1,), (1,)), ((), ())),
                        preferred_element_type=jnp.float32)
    yl_ref[...] = y[:, :HALF]
    yr_ref[...] = y[:, HALF:]
    r2_ref[...] = lax.dot_general(
        h, wroot_ref[...].astype(jnp.bfloat16), (((1,), (1,)), ((), ())),
        preferred_element_type=jnp.float32) + b_ref[...]


def _dense2(r, al, ar, W_rel, W_root, b):
    return pl.pallas_call(
        _dense2_body,
        grid=(N_BLKS,),
        in_specs=[
            pl.BlockSpec((ROW_BLK, D), lambda i: (i, 0)),
            pl.BlockSpec((ROW_BLK, HALF), lambda i: (i, 0)),
            pl.BlockSpec((ROW_BLK, HALF), lambda i: (i, 0)),
            pl.BlockSpec((D, D), lambda i: (0, 0)),
            pl.BlockSpec((D, D), lambda i: (0, 0)),
            pl.BlockSpec((1, D), lambda i: (0, 0)),
        ],
        out_specs=[
            pl.BlockSpec((ROW_BLK, HALF), lambda i: (i, 0)),
            pl.BlockSpec((ROW_BLK, HALF), lambda i: (i, 0)),
            pl.BlockSpec((ROW_BLK, D), lambda i: (i, 0)),
        ],
        out_shape=[
            jax.ShapeDtypeStruct((N, HALF), jnp.float32),
            jax.ShapeDtypeStruct((N, HALF), jnp.float32),
            jax.ShapeDtypeStruct((N, D), jnp.float32),
        ],
    )(r, al, ar, W_rel, W_root, b)


def _pool_body(r_ref, al_ref, ar_ref, batch_ref, o_ref):
    i = pl.program_id(0)
    h = jnp.concatenate([al_ref[...], ar_ref[...]], axis=1) + r_ref[...]
    h = jnp.maximum(h, 0.0)
    b = batch_ref[0, 0, :]
    mask = (b[:, None] == lax.broadcasted_iota(jnp.int32, (ROW_BLK, G), 1)
            ).astype(jnp.bfloat16)
    contrib = lax.dot_general(mask, h.astype(jnp.bfloat16),
                              (((0,), (0,)), ((), ())),
                              preferred_element_type=jnp.float32)

    @pl.when(i == 0)
    def _():
        o_ref[...] = contrib

    @pl.when(i != 0)
    def _():
        o_ref[...] += contrib


def _pool(r, al, ar, batch3):
    return pl.pallas_call(
        _pool_body,
        grid=(N_BLKS,),
        in_specs=[
            pl.BlockSpec((ROW_BLK, D), lambda i: (i, 0)),
            pl.BlockSpec((ROW_BLK, HALF), lambda i: (i, 0)),
            pl.BlockSpec((ROW_BLK, HALF), lambda i: (i, 0)),
            pl.BlockSpec((1, 1, ROW_BLK), lambda i: (i, 0, 0)),
        ],
        out_specs=pl.BlockSpec((G, D), lambda i: (0, 0)),
        out_shape=jax.ShapeDtypeStruct((G, D), jnp.float32),
    )(r, al, ar, batch3)


SEG_A = 64
SEG_B = CHUNKS_PER_TILE - SEG_A


def _sc_scatter_body(yl_hbm, yr_hbm, src_hbm, dst_hbm, al_hbm, ar_hbm,
                     acc, b0, b1, b2, src2v, dst2v,
                     gs0, gs1, gs2, ss0, ss1, ss2):
    c = lax.axis_index("c")
    t = lax.axis_index("s")
    bufs = (b0, b1, b2)
    gs = (gs0, gs1, gs2)
    ss = (ss0, ss1, ss2)

    pltpu.sync_copy(src_hbm.at[t, pl.ds(0, SEG_A)], src2v)
    pltpu.sync_copy(dst_hbm.at[t, pl.ds(0, SEG_A)], dst2v)

    zero16 = jnp.zeros((16,), jnp.float32)

    def zrow(i, carry):
        for k in range(HALF // 16):
            b0[i, pl.ds(k * 16, 16)] = zero16
            b1[i, pl.ds(k * 16, 16)] = zero16
        return carry

    lax.fori_loop(0, EDGE_CHUNK, zrow, 0)

    row0 = t * ROWS_PER_TILE
    for bi in range(ROWS_PER_TILE // EDGE_CHUNK):
        pltpu.sync_copy(b0 if bi % 2 == 0 else b1,
                        acc.at[pl.ds(row0 + bi * EDGE_CHUNK, EDGE_CHUNK)])
    pltpu.sync_copy(b1.at[pl.ds(0, ROWS_PER_TILE % EDGE_CHUNK)],
                    acc.at[pl.ds(row0 + 560, ROWS_PER_TILE % EDGE_CHUNK)])

    @pl.when(t == NUM_TILES - 1)
    def _():
        pltpu.sync_copy(b0.at[pl.ds(0, TAIL_ROWS)],
                        acc.at[pl.ds(NUM_TILES * ROWS_PER_TILE, TAIL_ROWS)])

    plsc.subcore_barrier()

    def gath(l, b):
        idx = src2v.at[l]

        @pl.when(c == 0)
        def _():
            pltpu.async_copy(yl_hbm.at[idx], bufs[b], gs[b])

        @pl.when(c == 1)
        def _():
            pltpu.async_copy(yr_hbm.at[idx], bufs[b], gs[b])

    def gw(b):
        pltpu.make_async_copy(
            yl_hbm.at[src2v.at[0]], bufs[b], gs[b]).wait()

    def sf(l, b):
        pltpu.async_copy(bufs[b], acc.at[dst2v.at[l]], ss[b], add=True)

    def sw(b):
        pltpu.make_async_copy(bufs[b], acc.at[dst2v.at[0]], ss[b]).wait()

    def step(l, b, refire):
        gw(b)
        sf(l, b)
        if refire:
            bp = (b + 2) % 3
            sw(bp)
            gath(l + 2, bp)

    def run_segment(n, n_groups, tail_refire_steps, tail_steps):
        gath(0, 0)
        gath(1, 1)
        gath(2, 2)
        step(0, 0, False)

        def grp(g, carry):
            l = 3 * g
            step(l + 1, 1, True)
            step(l + 2, 2, True)
            step(l + 3, 0, True)
            return carry

        lax.fori_loop(0, n_groups, grp, 0)
        for l in tail_refire_steps:
            step(l, l % 3, True)
        for l in tail_steps:
            step(l, l % 3, False)
        sw(0)
        sw(1)
        sw(2)

    run_segment(SEG_A, 20, [61], [62, 63])

    pltpu.sync_copy(src_hbm.at[t, pl.ds(SEG_A, SEG_B)],
                    src2v.at[pl.ds(0, SEG_B)])
    pltpu.sync_copy(dst_hbm.at[t, pl.ds(SEG_A, SEG_B)],
                    dst2v.at[pl.ds(0, SEG_B)])

    run_segment(SEG_B, 19, [58], [59, 60])

    plsc.subcore_barrier()

    tail0 = NUM_TILES * ROWS_PER_TILE

    @pl.when(c == 0)
    def _():
        pltpu.sync_copy(acc.at[pl.ds(row0, ROWS_PER_TILE)],
                        al_hbm.at[pl.ds(row0, ROWS_PER_TILE)])

        @pl.when(t == NUM_TILES - 1)
        def _():
            pltpu.sync_copy(acc.at[pl.ds(tail0, TAIL_ROWS)],
                            al_hbm.at[pl.ds(tail0, TAIL_ROWS)])

    @pl.when(c == 1)
    def _():
        pltpu.sync_copy(acc.at[pl.ds(row0, ROWS_PER_TILE)],
                        ar_hbm.at[pl.ds(row0, ROWS_PER_TILE)])

        @pl.when(t == NUM_TILES - 1)
        def _():
            pltpu.sync_copy(acc.at[pl.ds(tail0, TAIL_ROWS)],
                            ar_hbm.at[pl.ds(tail0, TAIL_ROWS)])


_SC_MESH = plsc.VectorSubcoreMesh(core_axis_name="c", subcore_axis_name="s",
                                  num_cores=2, num_subcores=NUM_TILES)

_sc_scatter = pl.kernel(
    _sc_scatter_body,
    out_type=[
        jax.ShapeDtypeStruct((N, HALF), jnp.float32),
        jax.ShapeDtypeStruct((N, HALF), jnp.float32),
    ],
    mesh=_SC_MESH,
    scratch_types=[
        pltpu.VMEM_SHARED((N, HALF), jnp.float32),
        pltpu.VMEM((EDGE_CHUNK, HALF), jnp.float32),
        pltpu.VMEM((EDGE_CHUNK, HALF), jnp.float32),
        pltpu.VMEM((EDGE_CHUNK, HALF), jnp.float32),
        pltpu.VMEM((SEG_A, EDGE_CHUNK), jnp.int32),
        pltpu.VMEM((SEG_A, EDGE_CHUNK), jnp.int32),
        pltpu.SemaphoreType.DMA,
        pltpu.SemaphoreType.DMA,
        pltpu.SemaphoreType.DMA,
        pltpu.SemaphoreType.DMA,
        pltpu.SemaphoreType.DMA,
        pltpu.SemaphoreType.DMA,
    ],
)


def kernel(x, edge_index, batch, W1_rel, b1, W1_root, W2_rel, b2, W2_root):
    src = edge_index[0].astype(jnp.int32).reshape(
        NUM_TILES, CHUNKS_PER_TILE, EDGE_CHUNK)
    dst = edge_index[1].astype(jnp.int32).reshape(
        NUM_TILES, CHUNKS_PER_TILE, EDGE_CHUNK)
    b1r = b1.reshape(1, D)
    b2r = b2.reshape(1, D)
    batch3 = batch.astype(jnp.int32).reshape(N_BLKS, 1, ROW_BLK)

    yl1, yr1, r1 = _dense1(x, W1_rel, W1_root, b1r)
    al1, ar1 = _sc_scatter(yl1, yr1, src, dst)
    yl2, yr2, r2 = _dense2(r1, al1, ar1, W2_rel, W2_root, b2r)
    ql, qr = _sc_scatter(yl2, yr2, src, dst)
    return _pool(r2, ql, qr, batch3)
